# baseline (speedup 1.0000x reference)
import functools

import jax
import jax.numpy as jnp
from jax import lax
from jax.experimental import pallas as pl
from jax.experimental.pallas import tpu as pltpu
from jax.experimental.pallas import tpu_sc as plsc

N = 10000
E = 320000
D = 128
EPS = 1e-5

NC = 2
NS = 16
NW = NC * NS
EPT = E // NW
CHUNK = 128
NCHUNK = 80
PCH = 40
EPTP = NCHUNK * CHUNK
MROWS = N + 8
RPTA = 624
TAIL = N - NS * RPTA
ZROWS = 48


def _sc_partial_agg(m, src3, dst3):
    mesh = plsc.VectorSubcoreMesh(core_axis_name="c", subcore_axis_name="s")

    @functools.partial(
        pl.kernel,
        out_type=jax.ShapeDtypeStruct((NC * N, D), jnp.float32),
        mesh=mesh,
        scratch_types=[
            pltpu.VMEM((PCH, CHUNK), jnp.int32),
            pltpu.VMEM((PCH, CHUNK), jnp.int32),
            pltpu.VMEM((2, CHUNK, D), jnp.float32),
            pltpu.VMEM((ZROWS, D), jnp.float32),
            pltpu.VMEM_SHARED((N, D), jnp.float32),
            pltpu.SemaphoreType.DMA,
        ],
    )
    def k(m_hbm, src_hbm, dst_hbm, out_hbm, sidx, didx, rows, zbuf, agg, sem):
        c = lax.axis_index("c")
        s = lax.axis_index("s")
        wid = c * NS + s

        def _zrow(i, carry):
            for j in range(D // 16):
                zbuf[i, pl.ds(j * 16, 16)] = jnp.zeros((16,), jnp.float32)
            return carry
        lax.fori_loop(0, ZROWS, _zrow, 0)
        row0 = s * RPTA
        for q in range(RPTA // ZROWS):
            pltpu.sync_copy(zbuf, agg.at[pl.ds(row0 + q * ZROWS, ZROWS)])

        @pl.when(s == NS - 1)
        def _tail_zero():
            pltpu.sync_copy(zbuf.at[pl.ds(0, TAIL)],
                            agg.at[pl.ds(N - TAIL, TAIL)])
        plsc.subcore_barrier()

        r0 = rows.at[0]
        r1 = rows.at[1]
        for ph in range(NCHUNK // PCH):
            pltpu.sync_copy(src_hbm.at[wid, pl.ds(ph * PCH, PCH)], sidx)
            pltpu.sync_copy(dst_hbm.at[wid, pl.ds(ph * PCH, PCH)], didx)
            pltpu.async_copy(m_hbm.at[sidx.at[0]], r0, sem)

            def _pair(jj, carry):
                j0 = 2 * jj
                j1 = j0 + 1
                jn = jnp.minimum(j0 + 2, PCH - 1)
                pltpu.make_async_copy(m_hbm.at[sidx.at[j0]], r0, sem).wait()
                pltpu.async_copy(m_hbm.at[sidx.at[j1]], r1, sem)
                pltpu.sync_copy(r0, agg.at[didx.at[j0]], add=True)
                pltpu.make_async_copy(m_hbm.at[sidx.at[j1]], r1, sem).wait()
                pltpu.async_copy(m_hbm.at[sidx.at[jn]], r0, sem)
                pltpu.sync_copy(r1, agg.at[didx.at[j1]], add=True)
                return carry
            lax.fori_loop(0, PCH // 2, _pair, 0)
            pltpu.make_async_copy(m_hbm.at[sidx.at[0]], r0, sem).wait()
        plsc.subcore_barrier()

        pltpu.sync_copy(agg.at[pl.ds(row0, RPTA)],
                        out_hbm.at[pl.ds(c * N + row0, RPTA)])

        @pl.when(s == NS - 1)
        def _tail_out():
            pltpu.sync_copy(agg.at[pl.ds(N - TAIL, TAIL)],
                            out_hbm.at[pl.ds(c * N + N - TAIL, TAIL)])

    return k(m, src3, dst3)


def _dotT(a, b_ref):
    return lax.dot_general(a, b_ref[...], (((1,), (1,)), ((), ())),
                           preferred_element_type=jnp.float32)


def _tc_emb(x, W_emb, b_emb, Wg0, v0):
    def body(x_ref, we_ref, be_ref, wg_ref, v_ref, m_ref, l_ref):
        h = _dotT(x_ref[...], we_ref) + be_ref[...]
        m_ref[0:N, :] = _dotT(h, wg_ref)
        m_ref[N:MROWS, :] = jnp.zeros((MROWS - N, D), jnp.float32)
        l_ref[...] = jnp.dot(h, v_ref[...], preferred_element_type=jnp.float32)
    return pl.pallas_call(
        body,
        out_shape=(jax.ShapeDtypeStruct((MROWS, D), jnp.float32),
                   jax.ShapeDtypeStruct((N, 1), jnp.float32)),
    )(x, W_emb, b_emb, Wg0, v0)


def _bn_relu(p_ref, bg_ref, g_ref, be_ref):
    agg = p_ref[0] + p_ref[1] + bg_ref[...]
    mu = jnp.mean(agg, axis=0, keepdims=True)
    cen = agg - mu
    var = jnp.mean(cen * cen, axis=0, keepdims=True)
    return jnp.maximum(g_ref[...] * cen * lax.rsqrt(var + EPS) + be_ref[...],
                       0.0)


def _tc_layer(p, bg, g, be, Wg_next, v_next, l_prev):
    def body(p_ref, bg_ref, g_ref, be_ref, wg_ref, v_ref, lp_ref,
             m_ref, l_ref):
        h = _bn_relu(p_ref, bg_ref, g_ref, be_ref)
        m_ref[0:N, :] = _dotT(h, wg_ref)
        m_ref[N:MROWS, :] = jnp.zeros((MROWS - N, D), jnp.float32)
        l_ref[...] = lp_ref[...] + jnp.dot(h, v_ref[...],
                                           preferred_element_type=jnp.float32)
    return pl.pallas_call(
        body,
        out_shape=(jax.ShapeDtypeStruct((MROWS, D), jnp.float32),
                   jax.ShapeDtypeStruct((N, 1), jnp.float32)),
    )(p, bg, g, be, Wg_next, v_next, l_prev)


def _tc_final(p, bg, g, be, v_last, l_prev):
    def body(p_ref, bg_ref, g_ref, be_ref, v_ref, lp_ref, lo_ref, sg_ref):
        h = _bn_relu(p_ref, bg_ref, g_ref, be_ref)
        logits = lp_ref[...] + jnp.dot(h, v_ref[...],
                                       preferred_element_type=jnp.float32)
        lo_ref[...] = logits
        sg_ref[...] = jax.nn.sigmoid(logits)
    return pl.pallas_call(
        body,
        out_shape=(jax.ShapeDtypeStruct((N, 1), jnp.float32),
                   jax.ShapeDtypeStruct((N, 1), jnp.float32)),
    )(p, bg, g, be, v_last, l_prev)


def kernel(x, edge_index, W_emb, b_emb, Wg0, Wg1, Wg2, Wg3,
           bg0, bg1, bg2, bg3, g0, g1, g2, g3, be0, be1, be2, be3,
           sk0, sk1, sk2, sk3, W_read):
    src_pad = jnp.full((NW, EPTP - EPT), N, jnp.int32)
    dst_pad = jnp.zeros((NW, EPTP - EPT), jnp.int32)
    src3 = jnp.concatenate([edge_index[0].reshape(NW, EPT), src_pad],
                           axis=1).reshape(NW, NCHUNK, CHUNK)
    dst3 = jnp.concatenate([edge_index[1].reshape(NW, EPT), dst_pad],
                           axis=1).reshape(NW, NCHUNK, CHUNK)

    w = W_read.reshape(4, D)
    v = [(w[i][None, :] @ [sk0, sk1, sk2, sk3][i]).reshape(D, 1)
         for i in range(4)]

    r1 = lambda a: a.reshape(1, D)
    m0, l0 = _tc_emb(x, W_emb, r1(b_emb), Wg0, v[0])
    p0 = _sc_partial_agg(m0, src3, dst3).reshape(NC, N, D)
    m1, l1 = _tc_layer(p0, r1(bg0), r1(g0), r1(be0), Wg1, v[1], l0)
    p1 = _sc_partial_agg(m1, src3, dst3).reshape(NC, N, D)
    m2, l2 = _tc_layer(p1, r1(bg1), r1(g1), r1(be1), Wg2, v[2], l1)
    p2 = _sc_partial_agg(m2, src3, dst3).reshape(NC, N, D)
    logits, sig = _tc_final(p2, r1(bg2), r1(g2), r1(be2), v[3], l2)
    return logits.reshape(-1), sig.reshape(-1)


# speedup vs baseline: 1.2294x; 1.2294x over previous
import functools

import jax
import jax.numpy as jnp
from jax import lax
from jax.experimental import pallas as pl
from jax.experimental.pallas import tpu as pltpu
from jax.experimental.pallas import tpu_sc as plsc

N = 10000
E = 320000
D = 128
EPS = 1e-5

NC = 2
NS = 16
NW = NC * NS
EPT = E // NW
CHUNK = 128
NCHUNK = 80
PCH = 40
EPTP = NCHUNK * CHUNK
MROWS = N + 8
RPTA = 624
TAIL = N - NS * RPTA
ZROWS = 48


def _sc_partial_agg(m, src3, dst3):
    mesh = plsc.VectorSubcoreMesh(core_axis_name="c", subcore_axis_name="s")

    @functools.partial(
        pl.kernel,
        out_type=jax.ShapeDtypeStruct((NC * N, D), jnp.float32),
        mesh=mesh,
        scratch_types=[
            pltpu.VMEM((PCH, CHUNK), jnp.int32),
            pltpu.VMEM((PCH, CHUNK), jnp.int32),
            pltpu.VMEM((2, CHUNK, D), jnp.float32),
            pltpu.VMEM((ZROWS, D), jnp.float32),
            pltpu.VMEM_SHARED((N, D), jnp.float32),
            pltpu.SemaphoreType.DMA,
        ],
    )
    def k(m_hbm, src_hbm, dst_hbm, out_hbm, sidx, didx, rows, zbuf, agg, sem):
        c = lax.axis_index("c")
        s = lax.axis_index("s")
        wid = c * NS + s

        def _zrow(i, carry):
            for j in range(D // 16):
                zbuf[i, pl.ds(j * 16, 16)] = jnp.zeros((16,), jnp.float32)
            return carry
        lax.fori_loop(0, ZROWS, _zrow, 0)
        row0 = s * RPTA
        for q in range(RPTA // ZROWS):
            pltpu.sync_copy(zbuf, agg.at[pl.ds(row0 + q * ZROWS, ZROWS)])

        @pl.when(s == NS - 1)
        def _tail_zero():
            pltpu.sync_copy(zbuf.at[pl.ds(0, TAIL)],
                            agg.at[pl.ds(N - TAIL, TAIL)])
        plsc.subcore_barrier()

        r0 = rows.at[0]
        r1 = rows.at[1]
        for ph in range(NCHUNK // PCH):
            pltpu.sync_copy(src_hbm.at[wid, pl.ds(ph * PCH, PCH)], sidx)
            pltpu.sync_copy(dst_hbm.at[wid, pl.ds(ph * PCH, PCH)], didx)
            def _chunk(j, carry):
                pltpu.async_copy(m_hbm.at[sidx.at[j]], r0, sem).wait()
                pltpu.sync_copy(r0, agg.at[didx.at[j]], add=True)
                return carry
            lax.fori_loop(0, PCH, _chunk, 0)
        plsc.subcore_barrier()

        pltpu.sync_copy(agg.at[pl.ds(row0, RPTA)],
                        out_hbm.at[pl.ds(c * N + row0, RPTA)])

        @pl.when(s == NS - 1)
        def _tail_out():
            pltpu.sync_copy(agg.at[pl.ds(N - TAIL, TAIL)],
                            out_hbm.at[pl.ds(c * N + N - TAIL, TAIL)])

    return k(m, src3, dst3)


def _dotT(a, b_ref):
    return lax.dot_general(a, b_ref[...], (((1,), (1,)), ((), ())),
                           preferred_element_type=jnp.float32)


def _tc_emb(x, W_emb, b_emb, Wg0, v0):
    def body(x_ref, we_ref, be_ref, wg_ref, v_ref, m_ref, l_ref):
        h = _dotT(x_ref[...], we_ref) + be_ref[...]
        m_ref[0:N, :] = _dotT(h, wg_ref)
        m_ref[N:MROWS, :] = jnp.zeros((MROWS - N, D), jnp.float32)
        l_ref[...] = jnp.dot(h, v_ref[...], preferred_element_type=jnp.float32)
    return pl.pallas_call(
        body,
        out_shape=(jax.ShapeDtypeStruct((MROWS, D), jnp.float32),
                   jax.ShapeDtypeStruct((N, 1), jnp.float32)),
    )(x, W_emb, b_emb, Wg0, v0)


def _bn_relu(p_ref, bg_ref, g_ref, be_ref):
    agg = p_ref[0] + p_ref[1] + bg_ref[...]
    mu = jnp.mean(agg, axis=0, keepdims=True)
    cen = agg - mu
    var = jnp.mean(cen * cen, axis=0, keepdims=True)
    return jnp.maximum(g_ref[...] * cen * lax.rsqrt(var + EPS) + be_ref[...],
                       0.0)


def _tc_layer(p, bg, g, be, Wg_next, v_next, l_prev):
    def body(p_ref, bg_ref, g_ref, be_ref, wg_ref, v_ref, lp_ref,
             m_ref, l_ref):
        h = _bn_relu(p_ref, bg_ref, g_ref, be_ref)
        m_ref[0:N, :] = _dotT(h, wg_ref)
        m_ref[N:MROWS, :] = jnp.zeros((MROWS - N, D), jnp.float32)
        l_ref[...] = lp_ref[...] + jnp.dot(h, v_ref[...],
                                           preferred_element_type=jnp.float32)
    return pl.pallas_call(
        body,
        out_shape=(jax.ShapeDtypeStruct((MROWS, D), jnp.float32),
                   jax.ShapeDtypeStruct((N, 1), jnp.float32)),
    )(p, bg, g, be, Wg_next, v_next, l_prev)


def _tc_final(p, bg, g, be, v_last, l_prev):
    def body(p_ref, bg_ref, g_ref, be_ref, v_ref, lp_ref, lo_ref, sg_ref):
        h = _bn_relu(p_ref, bg_ref, g_ref, be_ref)
        logits = lp_ref[...] + jnp.dot(h, v_ref[...],
                                       preferred_element_type=jnp.float32)
        lo_ref[...] = logits
        sg_ref[...] = jax.nn.sigmoid(logits)
    return pl.pallas_call(
        body,
        out_shape=(jax.ShapeDtypeStruct((N, 1), jnp.float32),
                   jax.ShapeDtypeStruct((N, 1), jnp.float32)),
    )(p, bg, g, be, v_last, l_prev)


def kernel(x, edge_index, W_emb, b_emb, Wg0, Wg1, Wg2, Wg3,
           bg0, bg1, bg2, bg3, g0, g1, g2, g3, be0, be1, be2, be3,
           sk0, sk1, sk2, sk3, W_read):
    src_pad = jnp.full((NW, EPTP - EPT), N, jnp.int32)
    dst_pad = jnp.zeros((NW, EPTP - EPT), jnp.int32)
    src3 = jnp.concatenate([edge_index[0].reshape(NW, EPT), src_pad],
                           axis=1).reshape(NW, NCHUNK, CHUNK)
    dst3 = jnp.concatenate([edge_index[1].reshape(NW, EPT), dst_pad],
                           axis=1).reshape(NW, NCHUNK, CHUNK)

    w = W_read.reshape(4, D)
    v = [(w[i][None, :] @ [sk0, sk1, sk2, sk3][i]).reshape(D, 1)
         for i in range(4)]

    r1 = lambda a: a.reshape(1, D)
    m0, l0 = _tc_emb(x, W_emb, r1(b_emb), Wg0, v[0])
    p0 = _sc_partial_agg(m0, src3, dst3).reshape(NC, N, D)
    m1, l1 = _tc_layer(p0, r1(bg0), r1(g0), r1(be0), Wg1, v[1], l0)
    p1 = _sc_partial_agg(m1, src3, dst3).reshape(NC, N, D)
    m2, l2 = _tc_layer(p1, r1(bg1), r1(g1), r1(be1), Wg2, v[2], l1)
    p2 = _sc_partial_agg(m2, src3, dst3).reshape(NC, N, D)
    logits, sig = _tc_final(p2, r1(bg2), r1(g2), r1(be2), v[3], l2)
    return logits.reshape(-1), sig.reshape(-1)


# speedup vs baseline: 1.2301x; 1.0005x over previous
import functools

import jax
import jax.numpy as jnp
from jax import lax
from jax.experimental import pallas as pl
from jax.experimental.pallas import tpu as pltpu
from jax.experimental.pallas import tpu_sc as plsc

N = 10000
E = 320000
D = 128
EPS = 1e-5

NC = 2
NS = 16
NW = NC * NS
EPT = E // NW
CHUNK = 128
NCHUNK = 80
PCH = 40
EPTP = NCHUNK * CHUNK
MROWS = N + 8
RPTA = 624
TAIL = N - NS * RPTA
ZROWS = 48


def _sc_partial_agg(m, src3, dst3):
    mesh = plsc.VectorSubcoreMesh(core_axis_name="c", subcore_axis_name="s")

    @functools.partial(
        pl.kernel,
        out_type=jax.ShapeDtypeStruct((NC * N, D), jnp.float32),
        mesh=mesh,
        scratch_types=[
            pltpu.VMEM((PCH, CHUNK), jnp.int32),
            pltpu.VMEM((PCH, CHUNK), jnp.int32),
            pltpu.VMEM((CHUNK, D), jnp.float32),
            pltpu.VMEM((ZROWS, D), jnp.float32),
            pltpu.VMEM_SHARED((N, D), jnp.float32),
            pltpu.SemaphoreType.DMA,
        ],
    )
    def k(m_hbm, src_hbm, dst_hbm, out_hbm, sidx, didx, rows, zbuf, agg, sem):
        c = lax.axis_index("c")
        s = lax.axis_index("s")
        wid = c * NS + s

        def _zrow(i, carry):
            for j in range(D // 16):
                zbuf[i, pl.ds(j * 16, 16)] = jnp.zeros((16,), jnp.float32)
            return carry
        lax.fori_loop(0, ZROWS, _zrow, 0)
        row0 = s * RPTA
        for q in range(RPTA // ZROWS):
            pltpu.sync_copy(zbuf, agg.at[pl.ds(row0 + q * ZROWS, ZROWS)])

        @pl.when(s == NS - 1)
        def _tail_zero():
            pltpu.sync_copy(zbuf.at[pl.ds(0, TAIL)],
                            agg.at[pl.ds(N - TAIL, TAIL)])
        plsc.subcore_barrier()

        r0 = rows
        for ph in range(NCHUNK // PCH):
            pltpu.sync_copy(src_hbm.at[wid, pl.ds(ph * PCH, PCH)], sidx)
            pltpu.sync_copy(dst_hbm.at[wid, pl.ds(ph * PCH, PCH)], didx)
            def _chunk(j, carry):
                pltpu.async_copy(m_hbm.at[sidx.at[j]], r0, sem).wait()
                pltpu.sync_copy(r0, agg.at[didx.at[j]], add=True)
                return carry
            lax.fori_loop(0, PCH, _chunk, 0)
        plsc.subcore_barrier()

        pltpu.sync_copy(agg.at[pl.ds(row0, RPTA)],
                        out_hbm.at[pl.ds(c * N + row0, RPTA)])

        @pl.when(s == NS - 1)
        def _tail_out():
            pltpu.sync_copy(agg.at[pl.ds(N - TAIL, TAIL)],
                            out_hbm.at[pl.ds(c * N + N - TAIL, TAIL)])

    return k(m, src3, dst3)


def _dotT(a, b_ref):
    return lax.dot_general(a, b_ref[...], (((1,), (1,)), ((), ())),
                           preferred_element_type=jnp.float32)


def _tc_emb(x, W_emb, b_emb, Wg0, v0):
    def body(x_ref, we_ref, be_ref, wg_ref, v_ref, m_ref, l_ref):
        h = _dotT(x_ref[...], we_ref) + be_ref[...]
        m_ref[0:N, :] = _dotT(h, wg_ref)
        m_ref[N:MROWS, :] = jnp.zeros((MROWS - N, D), jnp.float32)
        l_ref[...] = jnp.dot(h, v_ref[...], preferred_element_type=jnp.float32)
    return pl.pallas_call(
        body,
        out_shape=(jax.ShapeDtypeStruct((MROWS, D), jnp.float32),
                   jax.ShapeDtypeStruct((N, 1), jnp.float32)),
    )(x, W_emb, b_emb, Wg0, v0)


def _bn_relu(p_ref, bg_ref, g_ref, be_ref):
    agg = p_ref[0] + p_ref[1] + bg_ref[...]
    mu = jnp.mean(agg, axis=0, keepdims=True)
    cen = agg - mu
    var = jnp.mean(cen * cen, axis=0, keepdims=True)
    return jnp.maximum(g_ref[...] * cen * lax.rsqrt(var + EPS) + be_ref[...],
                       0.0)


def _tc_layer(p, bg, g, be, Wg_next, v_next, l_prev):
    def body(p_ref, bg_ref, g_ref, be_ref, wg_ref, v_ref, lp_ref,
             m_ref, l_ref):
        h = _bn_relu(p_ref, bg_ref, g_ref, be_ref)
        m_ref[0:N, :] = _dotT(h, wg_ref)
        m_ref[N:MROWS, :] = jnp.zeros((MROWS - N, D), jnp.float32)
        l_ref[...] = lp_ref[...] + jnp.dot(h, v_ref[...],
                                           preferred_element_type=jnp.float32)
    return pl.pallas_call(
        body,
        out_shape=(jax.ShapeDtypeStruct((MROWS, D), jnp.float32),
                   jax.ShapeDtypeStruct((N, 1), jnp.float32)),
    )(p, bg, g, be, Wg_next, v_next, l_prev)


def _tc_final(p, bg, g, be, v_last, l_prev):
    def body(p_ref, bg_ref, g_ref, be_ref, v_ref, lp_ref, lo_ref, sg_ref):
        h = _bn_relu(p_ref, bg_ref, g_ref, be_ref)
        logits = lp_ref[...] + jnp.dot(h, v_ref[...],
                                       preferred_element_type=jnp.float32)
        lo_ref[...] = logits
        sg_ref[...] = jax.nn.sigmoid(logits)
    return pl.pallas_call(
        body,
        out_shape=(jax.ShapeDtypeStruct((N, 1), jnp.float32),
                   jax.ShapeDtypeStruct((N, 1), jnp.float32)),
    )(p, bg, g, be, v_last, l_prev)


def kernel(x, edge_index, W_emb, b_emb, Wg0, Wg1, Wg2, Wg3,
           bg0, bg1, bg2, bg3, g0, g1, g2, g3, be0, be1, be2, be3,
           sk0, sk1, sk2, sk3, W_read):
    src_pad = jnp.full((NW, EPTP - EPT), N, jnp.int32)
    dst_pad = jnp.zeros((NW, EPTP - EPT), jnp.int32)
    src3 = jnp.concatenate([edge_index[0].reshape(NW, EPT), src_pad],
                           axis=1).reshape(NW, NCHUNK, CHUNK)
    dst3 = jnp.concatenate([edge_index[1].reshape(NW, EPT), dst_pad],
                           axis=1).reshape(NW, NCHUNK, CHUNK)

    w = W_read.reshape(4, D)
    v = [(w[i][None, :] @ [sk0, sk1, sk2, sk3][i]).reshape(D, 1)
         for i in range(4)]

    r1 = lambda a: a.reshape(1, D)
    m0, l0 = _tc_emb(x, W_emb, r1(b_emb), Wg0, v[0])
    p0 = _sc_partial_agg(m0, src3, dst3).reshape(NC, N, D)
    m1, l1 = _tc_layer(p0, r1(bg0), r1(g0), r1(be0), Wg1, v[1], l0)
    p1 = _sc_partial_agg(m1, src3, dst3).reshape(NC, N, D)
    m2, l2 = _tc_layer(p1, r1(bg1), r1(g1), r1(be1), Wg2, v[2], l1)
    p2 = _sc_partial_agg(m2, src3, dst3).reshape(NC, N, D)
    logits, sig = _tc_final(p2, r1(bg2), r1(g2), r1(be2), v[3], l2)
    return logits.reshape(-1), sig.reshape(-1)


# speedup vs baseline: 1.2319x; 1.0015x over previous
import functools

import jax
import jax.numpy as jnp
from jax import lax
from jax.experimental import pallas as pl
from jax.experimental.pallas import tpu as pltpu
from jax.experimental.pallas import tpu_sc as plsc

N = 10000
E = 320000
D = 128
EPS = 1e-5

NC = 2
NS = 16
NW = NC * NS
EPT = E // NW
CHUNK = 128
NCHUNK = 80
PCH = 80
EPTP = NCHUNK * CHUNK
MROWS = N + 8
RPTA = 624
TAIL = N - NS * RPTA
ZROWS = 48


def _sc_partial_agg(m, src3, dst3):
    mesh = plsc.VectorSubcoreMesh(core_axis_name="c", subcore_axis_name="s")

    @functools.partial(
        pl.kernel,
        out_type=jax.ShapeDtypeStruct((NC * N, D), jnp.float32),
        mesh=mesh,
        scratch_types=[
            pltpu.VMEM((PCH, CHUNK), jnp.int32),
            pltpu.VMEM((PCH, CHUNK), jnp.int32),
            pltpu.VMEM((CHUNK, D), jnp.float32),
            pltpu.VMEM((ZROWS, D), jnp.float32),
            pltpu.VMEM_SHARED((N, D), jnp.float32),
            pltpu.SemaphoreType.DMA,
        ],
    )
    def k(m_hbm, src_hbm, dst_hbm, out_hbm, sidx, didx, rows, zbuf, agg, sem):
        c = lax.axis_index("c")
        s = lax.axis_index("s")
        wid = c * NS + s

        def _zrow(i, carry):
            for j in range(D // 16):
                zbuf[i, pl.ds(j * 16, 16)] = jnp.zeros((16,), jnp.float32)
            return carry
        lax.fori_loop(0, ZROWS, _zrow, 0)
        row0 = s * RPTA
        for q in range(RPTA // ZROWS):
            pltpu.sync_copy(zbuf, agg.at[pl.ds(row0 + q * ZROWS, ZROWS)])

        @pl.when(s == NS - 1)
        def _tail_zero():
            pltpu.sync_copy(zbuf.at[pl.ds(0, TAIL)],
                            agg.at[pl.ds(N - TAIL, TAIL)])
        plsc.subcore_barrier()

        r0 = rows
        for ph in range(NCHUNK // PCH):
            pltpu.sync_copy(src_hbm.at[wid, pl.ds(ph * PCH, PCH)], sidx)
            pltpu.sync_copy(dst_hbm.at[wid, pl.ds(ph * PCH, PCH)], didx)
            def _chunk(j, carry):
                pltpu.async_copy(m_hbm.at[sidx.at[j]], r0, sem).wait()
                pltpu.sync_copy(r0, agg.at[didx.at[j]], add=True)
                return carry
            lax.fori_loop(0, PCH, _chunk, 0)
        plsc.subcore_barrier()

        pltpu.sync_copy(agg.at[pl.ds(row0, RPTA)],
                        out_hbm.at[pl.ds(c * N + row0, RPTA)])

        @pl.when(s == NS - 1)
        def _tail_out():
            pltpu.sync_copy(agg.at[pl.ds(N - TAIL, TAIL)],
                            out_hbm.at[pl.ds(c * N + N - TAIL, TAIL)])

    return k(m, src3, dst3)


def _dotT(a, b_ref):
    return lax.dot_general(a, b_ref[...], (((1,), (1,)), ((), ())),
                           preferred_element_type=jnp.float32)


def _tc_emb(x, W_emb, b_emb, Wg0, v0):
    def body(x_ref, we_ref, be_ref, wg_ref, v_ref, m_ref, l_ref):
        h = _dotT(x_ref[...], we_ref) + be_ref[...]
        m_ref[0:N, :] = _dotT(h, wg_ref)
        m_ref[N:MROWS, :] = jnp.zeros((MROWS - N, D), jnp.float32)
        l_ref[...] = jnp.dot(h, v_ref[...], preferred_element_type=jnp.float32)
    return pl.pallas_call(
        body,
        out_shape=(jax.ShapeDtypeStruct((MROWS, D), jnp.float32),
                   jax.ShapeDtypeStruct((N, 1), jnp.float32)),
    )(x, W_emb, b_emb, Wg0, v0)


def _bn_relu(p_ref, bg_ref, g_ref, be_ref):
    agg = p_ref[0] + p_ref[1] + bg_ref[...]
    mu = jnp.mean(agg, axis=0, keepdims=True)
    cen = agg - mu
    var = jnp.mean(cen * cen, axis=0, keepdims=True)
    return jnp.maximum(g_ref[...] * cen * lax.rsqrt(var + EPS) + be_ref[...],
                       0.0)


def _tc_layer(p, bg, g, be, Wg_next, v_next, l_prev):
    def body(p_ref, bg_ref, g_ref, be_ref, wg_ref, v_ref, lp_ref,
             m_ref, l_ref):
        h = _bn_relu(p_ref, bg_ref, g_ref, be_ref)
        m_ref[0:N, :] = _dotT(h, wg_ref)
        m_ref[N:MROWS, :] = jnp.zeros((MROWS - N, D), jnp.float32)
        l_ref[...] = lp_ref[...] + jnp.dot(h, v_ref[...],
                                           preferred_element_type=jnp.float32)
    return pl.pallas_call(
        body,
        out_shape=(jax.ShapeDtypeStruct((MROWS, D), jnp.float32),
                   jax.ShapeDtypeStruct((N, 1), jnp.float32)),
    )(p, bg, g, be, Wg_next, v_next, l_prev)


def _tc_final(p, bg, g, be, v_last, l_prev):
    def body(p_ref, bg_ref, g_ref, be_ref, v_ref, lp_ref, lo_ref, sg_ref):
        h = _bn_relu(p_ref, bg_ref, g_ref, be_ref)
        logits = lp_ref[...] + jnp.dot(h, v_ref[...],
                                       preferred_element_type=jnp.float32)
        lo_ref[...] = logits
        sg_ref[...] = jax.nn.sigmoid(logits)
    return pl.pallas_call(
        body,
        out_shape=(jax.ShapeDtypeStruct((N, 1), jnp.float32),
                   jax.ShapeDtypeStruct((N, 1), jnp.float32)),
    )(p, bg, g, be, v_last, l_prev)


def kernel(x, edge_index, W_emb, b_emb, Wg0, Wg1, Wg2, Wg3,
           bg0, bg1, bg2, bg3, g0, g1, g2, g3, be0, be1, be2, be3,
           sk0, sk1, sk2, sk3, W_read):
    src_pad = jnp.full((NW, EPTP - EPT), N, jnp.int32)
    dst_pad = jnp.tile(jnp.arange(EPTP - EPT, dtype=jnp.int32)[None, :],
                       (NW, 1))
    src3 = jnp.concatenate([edge_index[0].reshape(NW, EPT), src_pad],
                           axis=1).reshape(NW, NCHUNK, CHUNK)
    dst3 = jnp.concatenate([edge_index[1].reshape(NW, EPT), dst_pad],
                           axis=1).reshape(NW, NCHUNK, CHUNK)

    w = W_read.reshape(4, D)
    v = [(w[i][None, :] @ [sk0, sk1, sk2, sk3][i]).reshape(D, 1)
         for i in range(4)]

    r1 = lambda a: a.reshape(1, D)
    m0, l0 = _tc_emb(x, W_emb, r1(b_emb), Wg0, v[0])
    p0 = _sc_partial_agg(m0, src3, dst3).reshape(NC, N, D)
    m1, l1 = _tc_layer(p0, r1(bg0), r1(g0), r1(be0), Wg1, v[1], l0)
    p1 = _sc_partial_agg(m1, src3, dst3).reshape(NC, N, D)
    m2, l2 = _tc_layer(p1, r1(bg1), r1(g1), r1(be1), Wg2, v[2], l1)
    p2 = _sc_partial_agg(m2, src3, dst3).reshape(NC, N, D)
    logits, sig = _tc_final(p2, r1(bg2), r1(g2), r1(be2), v[3], l2)
    return logits.reshape(-1), sig.reshape(-1)


# speedup vs baseline: 1.2331x; 1.0009x over previous
import functools

import jax
import jax.numpy as jnp
from jax import lax
from jax.experimental import pallas as pl
from jax.experimental.pallas import tpu as pltpu
from jax.experimental.pallas import tpu_sc as plsc

N = 10000
E = 320000
D = 128
EPS = 1e-5

NC = 2
NS = 16
NW = NC * NS
EPT = E // NW
CHUNK = 128
NCHUNK = 80
PCH = 80
EPTP = NCHUNK * CHUNK
MROWS = N + 8
RPTA = 624
TAIL = N - NS * RPTA
ZROWS = 48


def _sc_partial_agg(m, src3, dst3):
    mesh = plsc.VectorSubcoreMesh(core_axis_name="c", subcore_axis_name="s")

    @functools.partial(
        pl.kernel,
        out_type=jax.ShapeDtypeStruct((NC * N, D), jnp.float32),
        mesh=mesh,
        scratch_types=[
            pltpu.VMEM((PCH, CHUNK), jnp.int32),
            pltpu.VMEM((PCH, CHUNK), jnp.int32),
            pltpu.VMEM((CHUNK, D), jnp.float32),
            pltpu.VMEM((ZROWS, D), jnp.float32),
            pltpu.VMEM_SHARED((N, D), jnp.float32),
            pltpu.SemaphoreType.DMA,
        ],
    )
    def k(m_hbm, src_hbm, dst_hbm, out_hbm, sidx, didx, rows, zbuf, agg, sem):
        c = lax.axis_index("c")
        s = lax.axis_index("s")
        wid = c * NS + s
        pltpu.sync_copy(src_hbm.at[wid], sidx)
        pltpu.sync_copy(dst_hbm.at[wid], didx)

        def _zrow(i, carry):
            for j in range(D // 16):
                zbuf[i, pl.ds(j * 16, 16)] = jnp.zeros((16,), jnp.float32)
            return carry
        lax.fori_loop(0, ZROWS, _zrow, 0)
        row0 = s * RPTA
        for q in range(RPTA // ZROWS):
            pltpu.sync_copy(zbuf, agg.at[pl.ds(row0 + q * ZROWS, ZROWS)])

        @pl.when(s == NS - 1)
        def _tail_zero():
            pltpu.sync_copy(zbuf.at[pl.ds(0, TAIL)],
                            agg.at[pl.ds(N - TAIL, TAIL)])
        plsc.subcore_barrier()

        def _chunk(j, carry):
            pltpu.async_copy(m_hbm.at[sidx.at[j]], rows, sem).wait()
            pltpu.sync_copy(rows, agg.at[didx.at[j]], add=True)
            return carry
        lax.fori_loop(0, NCHUNK, _chunk, 0)
        plsc.subcore_barrier()

        pltpu.sync_copy(agg.at[pl.ds(row0, RPTA)],
                        out_hbm.at[pl.ds(c * N + row0, RPTA)])

        @pl.when(s == NS - 1)
        def _tail_out():
            pltpu.sync_copy(agg.at[pl.ds(N - TAIL, TAIL)],
                            out_hbm.at[pl.ds(c * N + N - TAIL, TAIL)])

    return k(m, src3, dst3)


def _dotT(a, b_ref):
    return lax.dot_general(a, b_ref[...], (((1,), (1,)), ((), ())),
                           preferred_element_type=jnp.float32)


def _tc_emb(x, W_emb, b_emb, Wg0, v0):
    def body(x_ref, we_ref, be_ref, wg_ref, v_ref, m_ref, l_ref):
        h = _dotT(x_ref[...], we_ref) + be_ref[...]
        m_ref[0:N, :] = _dotT(h, wg_ref)
        m_ref[N:MROWS, :] = jnp.zeros((MROWS - N, D), jnp.float32)
        l_ref[...] = jnp.dot(h, v_ref[...], preferred_element_type=jnp.float32)
    return pl.pallas_call(
        body,
        out_shape=(jax.ShapeDtypeStruct((MROWS, D), jnp.float32),
                   jax.ShapeDtypeStruct((N, 1), jnp.float32)),
    )(x, W_emb, b_emb, Wg0, v0)


def _bn_relu(p_ref, bg_ref, g_ref, be_ref):
    agg = p_ref[0] + p_ref[1] + bg_ref[...]
    mu = jnp.mean(agg, axis=0, keepdims=True)
    cen = agg - mu
    var = jnp.mean(cen * cen, axis=0, keepdims=True)
    return jnp.maximum(g_ref[...] * cen * lax.rsqrt(var + EPS) + be_ref[...],
                       0.0)


def _tc_layer(p, bg, g, be, Wg_next, v_next, l_prev):
    def body(p_ref, bg_ref, g_ref, be_ref, wg_ref, v_ref, lp_ref,
             m_ref, l_ref):
        h = _bn_relu(p_ref, bg_ref, g_ref, be_ref)
        m_ref[0:N, :] = _dotT(h, wg_ref)
        m_ref[N:MROWS, :] = jnp.zeros((MROWS - N, D), jnp.float32)
        l_ref[...] = lp_ref[...] + jnp.dot(h, v_ref[...],
                                           preferred_element_type=jnp.float32)
    return pl.pallas_call(
        body,
        out_shape=(jax.ShapeDtypeStruct((MROWS, D), jnp.float32),
                   jax.ShapeDtypeStruct((N, 1), jnp.float32)),
    )(p, bg, g, be, Wg_next, v_next, l_prev)


def _tc_final(p, bg, g, be, v_last, l_prev):
    def body(p_ref, bg_ref, g_ref, be_ref, v_ref, lp_ref, lo_ref, sg_ref):
        h = _bn_relu(p_ref, bg_ref, g_ref, be_ref)
        logits = lp_ref[...] + jnp.dot(h, v_ref[...],
                                       preferred_element_type=jnp.float32)
        lo_ref[...] = logits
        sg_ref[...] = jax.nn.sigmoid(logits)
    return pl.pallas_call(
        body,
        out_shape=(jax.ShapeDtypeStruct((N, 1), jnp.float32),
                   jax.ShapeDtypeStruct((N, 1), jnp.float32)),
    )(p, bg, g, be, v_last, l_prev)


def kernel(x, edge_index, W_emb, b_emb, Wg0, Wg1, Wg2, Wg3,
           bg0, bg1, bg2, bg3, g0, g1, g2, g3, be0, be1, be2, be3,
           sk0, sk1, sk2, sk3, W_read):
    src_pad = jnp.full((NW, EPTP - EPT), N, jnp.int32)
    dst_pad = jnp.tile(jnp.arange(EPTP - EPT, dtype=jnp.int32)[None, :],
                       (NW, 1))
    src3 = jnp.concatenate([edge_index[0].reshape(NW, EPT), src_pad],
                           axis=1).reshape(NW, NCHUNK, CHUNK)
    dst3 = jnp.concatenate([edge_index[1].reshape(NW, EPT), dst_pad],
                           axis=1).reshape(NW, NCHUNK, CHUNK)

    w = W_read.reshape(4, D)
    v = [(w[i][None, :] @ [sk0, sk1, sk2, sk3][i]).reshape(D, 1)
         for i in range(4)]

    r1 = lambda a: a.reshape(1, D)
    m0, l0 = _tc_emb(x, W_emb, r1(b_emb), Wg0, v[0])
    p0 = _sc_partial_agg(m0, src3, dst3).reshape(NC, N, D)
    m1, l1 = _tc_layer(p0, r1(bg0), r1(g0), r1(be0), Wg1, v[1], l0)
    p1 = _sc_partial_agg(m1, src3, dst3).reshape(NC, N, D)
    m2, l2 = _tc_layer(p1, r1(bg1), r1(g1), r1(be1), Wg2, v[2], l1)
    p2 = _sc_partial_agg(m2, src3, dst3).reshape(NC, N, D)
    logits, sig = _tc_final(p2, r1(bg2), r1(g2), r1(be2), v[3], l2)
    return logits.reshape(-1), sig.reshape(-1)


# speedup vs baseline: 1.8090x; 1.4671x over previous
import functools

import jax
import jax.numpy as jnp
from jax import lax
from jax.experimental import pallas as pl
from jax.experimental.pallas import tpu as pltpu
from jax.experimental.pallas import tpu_sc as plsc

N = 10000
E = 320000
D = 128
EPS = 1e-5

NC = 2
NS = 16
NW = NC * NS
EPT = E // NW
CHUNK = 128
NCHUNK = 79
PCH = NCHUNK
EPTP = NCHUNK * CHUNK
MROWS = N + 8
RPTA = 624
TAIL = N - NS * RPTA
ZROWS = 48


def _sc_partial_agg(m, src3, dst3):
    mesh = plsc.VectorSubcoreMesh(core_axis_name="c", subcore_axis_name="s")

    @functools.partial(
        pl.kernel,
        out_type=jax.ShapeDtypeStruct((NC * N, D), jnp.float32),
        mesh=mesh,
        scratch_types=[
            pltpu.VMEM((PCH, CHUNK), jnp.int32),
            pltpu.VMEM((PCH, CHUNK), jnp.int32),
            pltpu.VMEM((CHUNK, D), jnp.float32),
            pltpu.VMEM((ZROWS, D), jnp.float32),
            pltpu.VMEM_SHARED((N, D), jnp.float32),
            pltpu.SemaphoreType.DMA,
        ],
    )
    def k(m_hbm, src_hbm, dst_hbm, out_hbm, sidx, didx, rows, zbuf, agg, sem):
        c = lax.axis_index("c")
        s = lax.axis_index("s")
        wid = c * NS + s
        pltpu.sync_copy(src_hbm.at[wid], sidx)
        pltpu.sync_copy(dst_hbm.at[wid], didx)

        def _zrow(i, carry):
            for j in range(D // 16):
                zbuf[i, pl.ds(j * 16, 16)] = jnp.zeros((16,), jnp.float32)
            return carry
        lax.fori_loop(0, ZROWS, _zrow, 0)
        row0 = s * RPTA
        for q in range(RPTA // ZROWS):
            pltpu.sync_copy(zbuf, agg.at[pl.ds(row0 + q * ZROWS, ZROWS)])

        @pl.when(s == NS - 1)
        def _tail_zero():
            pltpu.sync_copy(zbuf.at[pl.ds(0, TAIL)],
                            agg.at[pl.ds(N - TAIL, TAIL)])
        plsc.subcore_barrier()

        def _chunk(j, carry):
            pltpu.async_copy(m_hbm.at[sidx.at[j]], rows, sem).wait()
            pltpu.sync_copy(rows, agg.at[didx.at[j]], add=True)
            return carry
        lax.fori_loop(0, NCHUNK, _chunk, 0)
        plsc.subcore_barrier()

        pltpu.sync_copy(agg.at[pl.ds(row0, RPTA)],
                        out_hbm.at[pl.ds(c * N + row0, RPTA)])

        @pl.when(s == NS - 1)
        def _tail_out():
            pltpu.sync_copy(agg.at[pl.ds(N - TAIL, TAIL)],
                            out_hbm.at[pl.ds(c * N + N - TAIL, TAIL)])

    return k(m, src3, dst3)


def _dotT(a, b_ref):
    return lax.dot_general(a, b_ref[...], (((1,), (1,)), ((), ())),
                           preferred_element_type=jnp.float32)


def _tc_emb(x, W_emb, b_emb, Wg0, v0):
    def body(x_ref, we_ref, be_ref, wg_ref, v_ref, m_ref, l_ref):
        h = _dotT(x_ref[...], we_ref) + be_ref[...]
        m_ref[0:N, :] = _dotT(h, wg_ref)
        m_ref[N:MROWS, :] = jnp.zeros((MROWS - N, D), jnp.float32)
        l_ref[...] = jnp.dot(h, v_ref[...], preferred_element_type=jnp.float32)
    return pl.pallas_call(
        body,
        out_shape=(jax.ShapeDtypeStruct((MROWS, D), jnp.float32),
                   jax.ShapeDtypeStruct((N, 1), jnp.float32)),
    )(x, W_emb, b_emb, Wg0, v0)


def _bn_relu(p_ref, bg_ref, g_ref, be_ref):
    agg = p_ref[0] + p_ref[1] + bg_ref[...]
    mu = jnp.mean(agg, axis=0, keepdims=True)
    cen = agg - mu
    var = jnp.mean(cen * cen, axis=0, keepdims=True)
    return jnp.maximum(g_ref[...] * cen * lax.rsqrt(var + EPS) + be_ref[...],
                       0.0)


def _tc_layer(p, bg, g, be, Wg_next, v_next, l_prev):
    def body(p_ref, bg_ref, g_ref, be_ref, wg_ref, v_ref, lp_ref,
             m_ref, l_ref):
        h = _bn_relu(p_ref, bg_ref, g_ref, be_ref)
        m_ref[0:N, :] = _dotT(h, wg_ref)
        m_ref[N:MROWS, :] = jnp.zeros((MROWS - N, D), jnp.float32)
        l_ref[...] = lp_ref[...] + jnp.dot(h, v_ref[...],
                                           preferred_element_type=jnp.float32)
    return pl.pallas_call(
        body,
        out_shape=(jax.ShapeDtypeStruct((MROWS, D), jnp.float32),
                   jax.ShapeDtypeStruct((N, 1), jnp.float32)),
    )(p, bg, g, be, Wg_next, v_next, l_prev)


def _tc_final(p, bg, g, be, v_last, l_prev):
    def body(p_ref, bg_ref, g_ref, be_ref, v_ref, lp_ref, lo_ref, sg_ref):
        h = _bn_relu(p_ref, bg_ref, g_ref, be_ref)
        logits = lp_ref[...] + jnp.dot(h, v_ref[...],
                                       preferred_element_type=jnp.float32)
        lo_ref[...] = logits
        sg_ref[...] = jax.nn.sigmoid(logits)
    return pl.pallas_call(
        body,
        out_shape=(jax.ShapeDtypeStruct((N, 1), jnp.float32),
                   jax.ShapeDtypeStruct((N, 1), jnp.float32)),
    )(p, bg, g, be, v_last, l_prev)


def kernel(x, edge_index, W_emb, b_emb, Wg0, Wg1, Wg2, Wg3,
           bg0, bg1, bg2, bg3, g0, g1, g2, g3, be0, be1, be2, be3,
           sk0, sk1, sk2, sk3, W_read):
    src_pad = jnp.full((NW, EPTP - EPT), N, jnp.int32)
    dst_pad = jnp.zeros((NW, EPTP - EPT), jnp.int32)
    src3 = jnp.concatenate([edge_index[0].reshape(NW, EPT), src_pad],
                           axis=1).reshape(NW, NCHUNK, CHUNK)
    dst3 = jnp.concatenate([edge_index[1].reshape(NW, EPT), dst_pad],
                           axis=1).reshape(NW, NCHUNK, CHUNK)

    w = W_read.reshape(4, D)
    v = [(w[i][None, :] @ [sk0, sk1, sk2, sk3][i]).reshape(D, 1)
         for i in range(4)]

    r1 = lambda a: a.reshape(1, D)
    m0, l0 = _tc_emb(x, W_emb, r1(b_emb), Wg0, v[0])
    p0 = _sc_partial_agg(m0, src3, dst3).reshape(NC, N, D)
    m1, l1 = _tc_layer(p0, r1(bg0), r1(g0), r1(be0), Wg1, v[1], l0)
    p1 = _sc_partial_agg(m1, src3, dst3).reshape(NC, N, D)
    m2, l2 = _tc_layer(p1, r1(bg1), r1(g1), r1(be1), Wg2, v[2], l1)
    p2 = _sc_partial_agg(m2, src3, dst3).reshape(NC, N, D)
    logits, sig = _tc_final(p2, r1(bg2), r1(g2), r1(be2), v[3], l2)
    return logits.reshape(-1), sig.reshape(-1)


# speedup vs baseline: 2.9400x; 1.6253x over previous
import functools

import jax
import jax.numpy as jnp
from jax import lax
from jax.experimental import pallas as pl
from jax.experimental.pallas import tpu as pltpu
from jax.experimental.pallas import tpu_sc as plsc

N = 10000
E = 320000
D = 128
EPS = 1e-5

NC = 2
NS = 16
NW = NC * NS
EPT = E // NW
CHUNK = 128
NCHUNK = 79
PCH = NCHUNK
EPTP = NCHUNK * CHUNK
MROWS = N + (EPTP - EPT)
RPTA = 624
TAIL = N - NS * RPTA
ZROWS = 48


def _sc_partial_agg(m, src3, dst3):
    mesh = plsc.VectorSubcoreMesh(core_axis_name="c", subcore_axis_name="s")

    @functools.partial(
        pl.kernel,
        out_type=jax.ShapeDtypeStruct((NC * N, D), jnp.float32),
        mesh=mesh,
        scratch_types=[
            pltpu.VMEM((PCH, CHUNK), jnp.int32),
            pltpu.VMEM((PCH, CHUNK), jnp.int32),
            pltpu.VMEM((CHUNK, D), jnp.float32),
            pltpu.VMEM((ZROWS, D), jnp.float32),
            pltpu.VMEM_SHARED((N, D), jnp.float32),
            pltpu.SemaphoreType.DMA,
        ],
    )
    def k(m_hbm, src_hbm, dst_hbm, out_hbm, sidx, didx, rows, zbuf, agg, sem):
        c = lax.axis_index("c")
        s = lax.axis_index("s")
        wid = c * NS + s
        pltpu.sync_copy(src_hbm.at[wid], sidx)
        pltpu.sync_copy(dst_hbm.at[wid], didx)

        def _zrow(i, carry):
            for j in range(D // 16):
                zbuf[i, pl.ds(j * 16, 16)] = jnp.zeros((16,), jnp.float32)
            return carry
        lax.fori_loop(0, ZROWS, _zrow, 0)
        row0 = s * RPTA
        for q in range(RPTA // ZROWS):
            pltpu.sync_copy(zbuf, agg.at[pl.ds(row0 + q * ZROWS, ZROWS)])

        @pl.when(s == NS - 1)
        def _tail_zero():
            pltpu.sync_copy(zbuf.at[pl.ds(0, TAIL)],
                            agg.at[pl.ds(N - TAIL, TAIL)])
        plsc.subcore_barrier()

        def _chunk(j, carry):
            pltpu.async_copy(m_hbm.at[sidx.at[j]], rows, sem).wait()
            pltpu.sync_copy(rows, agg.at[didx.at[j]], add=True)
            return carry
        lax.fori_loop(0, NCHUNK, _chunk, 0)
        plsc.subcore_barrier()

        pltpu.sync_copy(agg.at[pl.ds(row0, RPTA)],
                        out_hbm.at[pl.ds(c * N + row0, RPTA)])

        @pl.when(s == NS - 1)
        def _tail_out():
            pltpu.sync_copy(agg.at[pl.ds(N - TAIL, TAIL)],
                            out_hbm.at[pl.ds(c * N + N - TAIL, TAIL)])

    return k(m, src3, dst3)


def _dotT(a, b_ref):
    return lax.dot_general(a, b_ref[...], (((1,), (1,)), ((), ())),
                           preferred_element_type=jnp.float32)


def _tc_emb(x, W_emb, b_emb, Wg0, v0):
    def body(x_ref, we_ref, be_ref, wg_ref, v_ref, m_ref, l_ref):
        h = _dotT(x_ref[...], we_ref) + be_ref[...]
        m_ref[0:N, :] = _dotT(h, wg_ref)
        m_ref[N:MROWS, :] = jnp.zeros((MROWS - N, D), jnp.float32)
        l_ref[...] = jnp.dot(h, v_ref[...], preferred_element_type=jnp.float32)
    return pl.pallas_call(
        body,
        out_shape=(jax.ShapeDtypeStruct((MROWS, D), jnp.float32),
                   jax.ShapeDtypeStruct((N, 1), jnp.float32)),
    )(x, W_emb, b_emb, Wg0, v0)


def _bn_relu(p_ref, bg_ref, g_ref, be_ref):
    agg = p_ref[0] + p_ref[1] + bg_ref[...]
    mu = jnp.mean(agg, axis=0, keepdims=True)
    cen = agg - mu
    var = jnp.mean(cen * cen, axis=0, keepdims=True)
    return jnp.maximum(g_ref[...] * cen * lax.rsqrt(var + EPS) + be_ref[...],
                       0.0)


def _tc_layer(p, bg, g, be, Wg_next, v_next, l_prev):
    def body(p_ref, bg_ref, g_ref, be_ref, wg_ref, v_ref, lp_ref,
             m_ref, l_ref):
        h = _bn_relu(p_ref, bg_ref, g_ref, be_ref)
        m_ref[0:N, :] = _dotT(h, wg_ref)
        m_ref[N:MROWS, :] = jnp.zeros((MROWS - N, D), jnp.float32)
        l_ref[...] = lp_ref[...] + jnp.dot(h, v_ref[...],
                                           preferred_element_type=jnp.float32)
    return pl.pallas_call(
        body,
        out_shape=(jax.ShapeDtypeStruct((MROWS, D), jnp.float32),
                   jax.ShapeDtypeStruct((N, 1), jnp.float32)),
    )(p, bg, g, be, Wg_next, v_next, l_prev)


def _tc_final(p, bg, g, be, v_last, l_prev):
    def body(p_ref, bg_ref, g_ref, be_ref, v_ref, lp_ref, lo_ref, sg_ref):
        h = _bn_relu(p_ref, bg_ref, g_ref, be_ref)
        logits = lp_ref[...] + jnp.dot(h, v_ref[...],
                                       preferred_element_type=jnp.float32)
        lo_ref[...] = logits
        sg_ref[...] = jax.nn.sigmoid(logits)
    return pl.pallas_call(
        body,
        out_shape=(jax.ShapeDtypeStruct((N, 1), jnp.float32),
                   jax.ShapeDtypeStruct((N, 1), jnp.float32)),
    )(p, bg, g, be, v_last, l_prev)


def kernel(x, edge_index, W_emb, b_emb, Wg0, Wg1, Wg2, Wg3,
           bg0, bg1, bg2, bg3, g0, g1, g2, g3, be0, be1, be2, be3,
           sk0, sk1, sk2, sk3, W_read):
    src_pad = jnp.tile(N + jnp.arange(EPTP - EPT, dtype=jnp.int32)[None, :],
                       (NW, 1))
    dst_pad = jnp.tile(jnp.arange(EPTP - EPT, dtype=jnp.int32)[None, :],
                       (NW, 1))
    src3 = jnp.concatenate([edge_index[0].reshape(NW, EPT), src_pad],
                           axis=1).reshape(NW, NCHUNK, CHUNK)
    dst3 = jnp.concatenate([edge_index[1].reshape(NW, EPT), dst_pad],
                           axis=1).reshape(NW, NCHUNK, CHUNK)

    w = W_read.reshape(4, D)
    v = [(w[i][None, :] @ [sk0, sk1, sk2, sk3][i]).reshape(D, 1)
         for i in range(4)]

    r1 = lambda a: a.reshape(1, D)
    m0, l0 = _tc_emb(x, W_emb, r1(b_emb), Wg0, v[0])
    p0 = _sc_partial_agg(m0, src3, dst3).reshape(NC, N, D)
    m1, l1 = _tc_layer(p0, r1(bg0), r1(g0), r1(be0), Wg1, v[1], l0)
    p1 = _sc_partial_agg(m1, src3, dst3).reshape(NC, N, D)
    m2, l2 = _tc_layer(p1, r1(bg1), r1(g1), r1(be1), Wg2, v[2], l1)
    p2 = _sc_partial_agg(m2, src3, dst3).reshape(NC, N, D)
    logits, sig = _tc_final(p2, r1(bg2), r1(g2), r1(be2), v[3], l2)
    return logits.reshape(-1), sig.reshape(-1)


# speedup vs baseline: 3.6937x; 1.2564x over previous
import functools

import jax
import jax.numpy as jnp
from jax import lax
from jax.experimental import pallas as pl
from jax.experimental.pallas import tpu as pltpu
from jax.experimental.pallas import tpu_sc as plsc

N = 10000
E = 320000
D = 128
EPS = 1e-5

NC = 2
NS = 16
NW = NC * NS
EPT = E // NW
CHUNK = 128
NCHUNK = 80
PCH = 40
EPTP = NCHUNK * CHUNK
MROWS = N + (EPTP - EPT)
RPTA = 624
TAIL = N - NS * RPTA
ZROWS = 48


def _sc_partial_agg(m, src3, dst3):
    mesh = plsc.VectorSubcoreMesh(core_axis_name="c", subcore_axis_name="s")

    @functools.partial(
        pl.kernel,
        out_type=jax.ShapeDtypeStruct((NC * N, D), jnp.float32),
        mesh=mesh,
        scratch_types=[
            pltpu.VMEM((PCH, CHUNK), jnp.int32),
            pltpu.VMEM((PCH, CHUNK), jnp.int32),
            pltpu.VMEM((CHUNK, D), jnp.float32),
            pltpu.VMEM((CHUNK, D), jnp.float32),
            pltpu.VMEM((ZROWS, D), jnp.float32),
            pltpu.VMEM_SHARED((N, D), jnp.float32),
            pltpu.SemaphoreType.DMA,
        ],
    )
    def k(m_hbm, src_hbm, dst_hbm, out_hbm, sidx, didx, ra, rb, zbuf, agg,
          sem):
        c = lax.axis_index("c")
        s = lax.axis_index("s")
        wid = c * NS + s

        def _zrow(i, carry):
            for j in range(D // 16):
                zbuf[i, pl.ds(j * 16, 16)] = jnp.zeros((16,), jnp.float32)
            return carry
        lax.fori_loop(0, ZROWS, _zrow, 0)
        row0 = s * RPTA
        for q in range(RPTA // ZROWS):
            pltpu.sync_copy(zbuf, agg.at[pl.ds(row0 + q * ZROWS, ZROWS)])

        @pl.when(s == NS - 1)
        def _tail_zero():
            pltpu.sync_copy(zbuf.at[pl.ds(0, TAIL)],
                            agg.at[pl.ds(N - TAIL, TAIL)])
        plsc.subcore_barrier()

        for ph in range(NCHUNK // PCH):
            pltpu.sync_copy(src_hbm.at[wid, pl.ds(ph * PCH, PCH)], sidx)
            pltpu.sync_copy(dst_hbm.at[wid, pl.ds(ph * PCH, PCH)], didx)
            pltpu.async_copy(m_hbm.at[sidx.at[0]], ra, sem)

            def _pair(jj, carry):
                j0 = 2 * jj
                j1 = j0 + 1
                jn = jnp.minimum(j0 + 2, PCH - 1)
                pltpu.make_async_copy(m_hbm.at[sidx.at[j0]], ra, sem).wait()
                pltpu.async_copy(m_hbm.at[sidx.at[j1]], rb, sem)
                pltpu.sync_copy(ra, agg.at[didx.at[j0]], add=True)
                pltpu.make_async_copy(m_hbm.at[sidx.at[j1]], rb, sem).wait()
                pltpu.async_copy(m_hbm.at[sidx.at[jn]], ra, sem)
                pltpu.sync_copy(rb, agg.at[didx.at[j1]], add=True)
                return carry
            lax.fori_loop(0, PCH // 2, _pair, 0)
            pltpu.make_async_copy(m_hbm.at[sidx.at[0]], ra, sem).wait()
        plsc.subcore_barrier()

        pltpu.sync_copy(agg.at[pl.ds(row0, RPTA)],
                        out_hbm.at[pl.ds(c * N + row0, RPTA)])

        @pl.when(s == NS - 1)
        def _tail_out():
            pltpu.sync_copy(agg.at[pl.ds(N - TAIL, TAIL)],
                            out_hbm.at[pl.ds(c * N + N - TAIL, TAIL)])

    return k(m, src3, dst3)


def _dotT(a, b_ref):
    return lax.dot_general(a, b_ref[...], (((1,), (1,)), ((), ())),
                           preferred_element_type=jnp.float32)


def _tc_emb(x, W_emb, b_emb, Wg0, v0):
    def body(x_ref, we_ref, be_ref, wg_ref, v_ref, m_ref, l_ref):
        h = _dotT(x_ref[...], we_ref) + be_ref[...]
        m_ref[0:N, :] = _dotT(h, wg_ref)
        m_ref[N:MROWS, :] = jnp.zeros((MROWS - N, D), jnp.float32)
        l_ref[...] = jnp.dot(h, v_ref[...], preferred_element_type=jnp.float32)
    return pl.pallas_call(
        body,
        out_shape=(jax.ShapeDtypeStruct((MROWS, D), jnp.float32),
                   jax.ShapeDtypeStruct((N, 1), jnp.float32)),
    )(x, W_emb, b_emb, Wg0, v0)


def _bn_relu(p_ref, bg_ref, g_ref, be_ref):
    agg = p_ref[0] + p_ref[1] + bg_ref[...]
    mu = jnp.mean(agg, axis=0, keepdims=True)
    cen = agg - mu
    var = jnp.mean(cen * cen, axis=0, keepdims=True)
    return jnp.maximum(g_ref[...] * cen * lax.rsqrt(var + EPS) + be_ref[...],
                       0.0)


def _tc_layer(p, bg, g, be, Wg_next, v_next, l_prev):
    def body(p_ref, bg_ref, g_ref, be_ref, wg_ref, v_ref, lp_ref,
             m_ref, l_ref):
        h = _bn_relu(p_ref, bg_ref, g_ref, be_ref)
        m_ref[0:N, :] = _dotT(h, wg_ref)
        m_ref[N:MROWS, :] = jnp.zeros((MROWS - N, D), jnp.float32)
        l_ref[...] = lp_ref[...] + jnp.dot(h, v_ref[...],
                                           preferred_element_type=jnp.float32)
    return pl.pallas_call(
        body,
        out_shape=(jax.ShapeDtypeStruct((MROWS, D), jnp.float32),
                   jax.ShapeDtypeStruct((N, 1), jnp.float32)),
    )(p, bg, g, be, Wg_next, v_next, l_prev)


def _tc_final(p, bg, g, be, v_last, l_prev):
    def body(p_ref, bg_ref, g_ref, be_ref, v_ref, lp_ref, lo_ref, sg_ref):
        h = _bn_relu(p_ref, bg_ref, g_ref, be_ref)
        logits = lp_ref[...] + jnp.dot(h, v_ref[...],
                                       preferred_element_type=jnp.float32)
        lo_ref[...] = logits
        sg_ref[...] = jax.nn.sigmoid(logits)
    return pl.pallas_call(
        body,
        out_shape=(jax.ShapeDtypeStruct((N, 1), jnp.float32),
                   jax.ShapeDtypeStruct((N, 1), jnp.float32)),
    )(p, bg, g, be, v_last, l_prev)


def kernel(x, edge_index, W_emb, b_emb, Wg0, Wg1, Wg2, Wg3,
           bg0, bg1, bg2, bg3, g0, g1, g2, g3, be0, be1, be2, be3,
           sk0, sk1, sk2, sk3, W_read):
    src_pad = jnp.tile(N + jnp.arange(EPTP - EPT, dtype=jnp.int32)[None, :],
                       (NW, 1))
    dst_pad = jnp.tile(jnp.arange(EPTP - EPT, dtype=jnp.int32)[None, :],
                       (NW, 1))
    src3 = jnp.concatenate([edge_index[0].reshape(NW, EPT), src_pad],
                           axis=1).reshape(NW, NCHUNK, CHUNK)
    dst3 = jnp.concatenate([edge_index[1].reshape(NW, EPT), dst_pad],
                           axis=1).reshape(NW, NCHUNK, CHUNK)

    w = W_read.reshape(4, D)
    v = [(w[i][None, :] @ [sk0, sk1, sk2, sk3][i]).reshape(D, 1)
         for i in range(4)]

    r1 = lambda a: a.reshape(1, D)
    m0, l0 = _tc_emb(x, W_emb, r1(b_emb), Wg0, v[0])
    p0 = _sc_partial_agg(m0, src3, dst3).reshape(NC, N, D)
    m1, l1 = _tc_layer(p0, r1(bg0), r1(g0), r1(be0), Wg1, v[1], l0)
    p1 = _sc_partial_agg(m1, src3, dst3).reshape(NC, N, D)
    m2, l2 = _tc_layer(p1, r1(bg1), r1(g1), r1(be1), Wg2, v[2], l1)
    p2 = _sc_partial_agg(m2, src3, dst3).reshape(NC, N, D)
    logits, sig = _tc_final(p2, r1(bg2), r1(g2), r1(be2), v[3], l2)
    return logits.reshape(-1), sig.reshape(-1)
